# baseline (speedup 1.0000x reference)
import functools

import jax
import jax.numpy as jnp
from jax import lax
from jax.experimental import pallas as pl
from jax.experimental.pallas import tpu as pltpu
from jax.experimental.pallas import tpu_sc as plsc

BATCH = 16384
EMBED_DIM = 32
ROW_W = 128
USERS_PER_ROW = ROW_W // EMBED_DIM
NUM_CORES = 2
NUM_SUBCORES = 16
LANES = 16
NUM_WORKERS = NUM_CORES * NUM_SUBCORES
BPW = BATCH // NUM_WORKERS
CHUNK = 128
NUM_CHUNKS = BPW // CHUNK
GROUPS_PER_CHUNK = CHUNK // LANES

_mesh = plsc.VectorSubcoreMesh(core_axis_name="c", subcore_axis_name="s")


@functools.partial(
    pl.kernel,
    out_type=jax.ShapeDtypeStruct((BATCH,), jnp.float32),
    mesh=_mesh,
    compiler_params=pltpu.CompilerParams(needs_layout_passes=False),
    scratch_types=[
        pltpu.VMEM((BPW,), jnp.int32),
        pltpu.VMEM((BPW,), jnp.int32),
        pltpu.VMEM((BPW,), jnp.int32),
        pltpu.VMEM((BPW,), jnp.int32),
        pltpu.VMEM((CHUNK, ROW_W), jnp.float32),
        pltpu.VMEM((CHUNK, ROW_W), jnp.float32),
        pltpu.VMEM((BPW,), jnp.float32),
        pltpu.SemaphoreType.DMA,
        pltpu.SemaphoreType.DMA,
    ],
)
def _sc_dot(uid_hbm, iid_hbm, utab_hbm, itab_hbm, out_hbm,
            uidx, iidx, urow, irow, uslab, islab, outb, sem_u, sem_i):
    wid = lax.axis_index("s") * NUM_CORES + lax.axis_index("c")
    base = wid * BPW

    pltpu.sync_copy(uid_hbm.at[pl.ds(base, BPW)], uidx)
    pltpu.sync_copy(iid_hbm.at[pl.ds(base, BPW)], iidx)

    def rowidx_body(i, carry):
        sl = pl.ds(i * LANES, LANES)
        urow[sl] = uidx[sl] // USERS_PER_ROW
        irow[sl] = iidx[sl] // USERS_PER_ROW
        return carry

    lax.fori_loop(0, BPW // LANES, rowidx_body, 0)

    lane = lax.iota(jnp.int32, LANES)

    def chunk_body(j, carry):
        csl = pl.ds(j * CHUNK, CHUNK)
        cu = pltpu.async_copy(utab_hbm.at[urow.at[csl]], uslab, sem_u)
        ci = pltpu.async_copy(itab_hbm.at[irow.at[csl]], islab, sem_i)
        cu.wait()
        ci.wait()

        def group_body(g, carry2):
            gsl = pl.ds(j * CHUNK + g * LANES, LANES)
            rvec = g * LANES + lane
            ubase = (uidx[gsl] % USERS_PER_ROW) * EMBED_DIM
            ibase = (iidx[gsl] % USERS_PER_ROW) * EMBED_DIM
            acc = jnp.zeros((LANES,), jnp.float32)
            for d in range(EMBED_DIM):
                uu = plsc.load_gather(uslab, [rvec, ubase + d])
                vv = plsc.load_gather(islab, [rvec, ibase + d])
                acc = acc + uu * vv
            outb[gsl] = acc
            return carry2

        lax.fori_loop(0, GROUPS_PER_CHUNK, group_body, 0)
        return carry

    lax.fori_loop(0, NUM_CHUNKS, chunk_body, 0)

    pltpu.sync_copy(outb, out_hbm.at[pl.ds(base, BPW)])


def kernel(user_ids, item_ids, user_table, item_table):
    nrows = user_table.shape[0] * EMBED_DIM // ROW_W
    ut = user_table.reshape(nrows, ROW_W)
    it = item_table.reshape(nrows, ROW_W)
    return _sc_dot(user_ids.astype(jnp.int32), item_ids.astype(jnp.int32),
                   ut, it)


# speedup vs baseline: 3.8275x; 3.8275x over previous
import functools

import jax
import jax.numpy as jnp
from jax import lax
from jax.experimental import pallas as pl
from jax.experimental.pallas import tpu as pltpu
from jax.experimental.pallas import tpu_sc as plsc

BATCH = 16384
EMBED_DIM = 32
LANE_TILE = 128
NUM_CORES = 2
NUM_SUBCORES = 16
LANES = 16
NUM_WORKERS = NUM_CORES * NUM_SUBCORES
BPW = BATCH // NUM_WORKERS
GROUPS = BPW // LANES
NRING = 8

_mesh = plsc.VectorSubcoreMesh(core_axis_name="c", subcore_axis_name="s")


@functools.partial(
    pl.kernel,
    out_type=jax.ShapeDtypeStruct((BATCH,), jnp.float32),
    mesh=_mesh,
    compiler_params=pltpu.CompilerParams(needs_layout_passes=False),
    scratch_types=[
        pltpu.VMEM((BPW,), jnp.int32),
        pltpu.VMEM((BPW,), jnp.int32),
        pltpu.VMEM((NRING, EMBED_DIM, LANE_TILE), jnp.float32),
        pltpu.VMEM((NRING, EMBED_DIM, LANE_TILE), jnp.float32),
        pltpu.VMEM((LANES, EMBED_DIM), jnp.float32),
        pltpu.VMEM((LANES, EMBED_DIM), jnp.float32),
        pltpu.VMEM((BPW,), jnp.float32),
        pltpu.SemaphoreType.DMA,
        pltpu.SemaphoreType.DMA,
    ],
)
def _sc_dot(uid_hbm, iid_hbm, utabT_hbm, itabT_hbm, out_hbm,
            uidx, iidx, uslabs, islabs, urows, irows, outb, sem_u, sem_i):
    wid = lax.axis_index("s") * NUM_CORES + lax.axis_index("c")
    base = wid * BPW

    pltpu.sync_copy(uid_hbm.at[pl.ds(base, BPW)], uidx)
    pltpu.sync_copy(iid_hbm.at[pl.ds(base, BPW)], iidx)

    dlo = lax.iota(jnp.int32, LANES)
    dhi = dlo + LANES

    def slab_src(tab, idv):
        off = pl.multiple_of((idv // LANE_TILE) * LANE_TILE, LANE_TILE)
        return tab.at[:, pl.ds(off, LANE_TILE)]

    def group_body(g, carry):
        uvec = uidx[pl.ds(g * LANES, LANES)]
        ivec = iidx[pl.ds(g * LANES, LANES)]

        for j in range(NRING):
            pltpu.async_copy(slab_src(utabT_hbm, uvec[j]), uslabs.at[j], sem_u)
            pltpu.async_copy(slab_src(itabT_hbm, ivec[j]), islabs.at[j], sem_i)

        for j in range(LANES):
            slot = j % NRING
            pltpu.make_async_copy(slab_src(utabT_hbm, uvec[j]),
                                  uslabs.at[slot], sem_u).wait()
            pltpu.make_async_copy(slab_src(itabT_hbm, ivec[j]),
                                  islabs.at[slot], sem_i).wait()
            uc = jnp.full((LANES,), uvec[j] % LANE_TILE, jnp.int32)
            ic = jnp.full((LANES,), ivec[j] % LANE_TILE, jnp.int32)
            urows[j, pl.ds(0, LANES)] = plsc.load_gather(uslabs.at[slot],
                                                         [dlo, uc])
            urows[j, pl.ds(LANES, LANES)] = plsc.load_gather(uslabs.at[slot],
                                                             [dhi, uc])
            irows[j, pl.ds(0, LANES)] = plsc.load_gather(islabs.at[slot],
                                                         [dlo, ic])
            irows[j, pl.ds(LANES, LANES)] = plsc.load_gather(islabs.at[slot],
                                                             [dhi, ic])
            if j + NRING < LANES:
                pltpu.async_copy(slab_src(utabT_hbm, uvec[j + NRING]),
                                 uslabs.at[slot], sem_u)
                pltpu.async_copy(slab_src(itabT_hbm, ivec[j + NRING]),
                                 islabs.at[slot], sem_i)

        acc = jnp.zeros((LANES,), jnp.float32)
        for d in range(EMBED_DIM):
            dv = jnp.full((LANES,), d, jnp.int32)
            acc = acc + (plsc.load_gather(urows, [dlo, dv]) *
                         plsc.load_gather(irows, [dlo, dv]))
        outb[pl.ds(g * LANES, LANES)] = acc
        return carry

    lax.fori_loop(0, GROUPS, group_body, 0)

    pltpu.sync_copy(outb, out_hbm.at[pl.ds(base, BPW)])


def kernel(user_ids, item_ids, user_table, item_table):
    return _sc_dot(user_ids.astype(jnp.int32), item_ids.astype(jnp.int32),
                   user_table.T, item_table.T)


# speedup vs baseline: 4.0584x; 1.0603x over previous
import functools

import jax
import jax.numpy as jnp
from jax import lax
from jax.experimental import pallas as pl
from jax.experimental.pallas import tpu as pltpu
from jax.experimental.pallas import tpu_sc as plsc

BATCH = 16384
EMBED_DIM = 32
LANE_TILE = 128
NUM_CORES = 2
NUM_SUBCORES = 16
LANES = 16
NUM_WORKERS = NUM_CORES * NUM_SUBCORES
BPW = BATCH // NUM_WORKERS
GROUPS = BPW // LANES
NRING = 8

_mesh = plsc.VectorSubcoreMesh(core_axis_name="c", subcore_axis_name="s")


@functools.partial(
    pl.kernel,
    out_type=jax.ShapeDtypeStruct((BATCH,), jnp.float32),
    mesh=_mesh,
    compiler_params=pltpu.CompilerParams(needs_layout_passes=False),
    scratch_types=[
        pltpu.VMEM((BPW + LANES,), jnp.int32),
        pltpu.VMEM((BPW + LANES,), jnp.int32),
        pltpu.VMEM((NRING, EMBED_DIM, LANE_TILE), jnp.float32),
        pltpu.VMEM((NRING, EMBED_DIM, LANE_TILE), jnp.float32),
        pltpu.VMEM((BPW,), jnp.float32),
        pltpu.SemaphoreType.DMA,
        pltpu.SemaphoreType.DMA,
    ],
)
def _sc_dot(uid_hbm, iid_hbm, utabT_hbm, itabT_hbm, out_hbm,
            uidx, iidx, uslabs, islabs, outb, sem_u, sem_i):
    wid = lax.axis_index("s") * NUM_CORES + lax.axis_index("c")
    base = wid * BPW

    pltpu.sync_copy(uid_hbm.at[pl.ds(base, BPW)], uidx.at[pl.ds(0, BPW)])
    pltpu.sync_copy(iid_hbm.at[pl.ds(base, BPW)], iidx.at[pl.ds(0, BPW)])
    uidx[pl.ds(BPW, LANES)] = jnp.zeros((LANES,), jnp.int32)
    iidx[pl.ds(BPW, LANES)] = jnp.zeros((LANES,), jnp.int32)

    dlo = lax.iota(jnp.int32, LANES)
    dhi = dlo + LANES

    def slab_src(tab, idv):
        off = pl.multiple_of((idv // LANE_TILE) * LANE_TILE, LANE_TILE)
        return tab.at[:, pl.ds(off, LANE_TILE)]

    uvec0 = uidx[pl.ds(0, LANES)]
    ivec0 = iidx[pl.ds(0, LANES)]
    for j in range(NRING):
        pltpu.async_copy(slab_src(utabT_hbm, uvec0[j]), uslabs.at[j], sem_u)
        pltpu.async_copy(slab_src(itabT_hbm, ivec0[j]), islabs.at[j], sem_i)

    onehot = [(lax.iota(jnp.int32, LANES) == j).astype(jnp.float32)
              for j in range(LANES)]

    def group_body(g, carry):
        uvec = uidx[pl.ds(g * LANES, LANES)]
        ivec = iidx[pl.ds(g * LANES, LANES)]
        uvecn = uidx[pl.ds(g * LANES + NRING, LANES)]
        ivecn = iidx[pl.ds(g * LANES + NRING, LANES)]

        acc = jnp.zeros((LANES,), jnp.float32)
        for j in range(LANES):
            slot = j % NRING
            pltpu.make_async_copy(slab_src(utabT_hbm, uvec[j]),
                                  uslabs.at[slot], sem_u).wait()
            pltpu.make_async_copy(slab_src(itabT_hbm, ivec[j]),
                                  islabs.at[slot], sem_i).wait()
            uc = jnp.full((LANES,), uvec[j] % LANE_TILE, jnp.int32)
            ic = jnp.full((LANES,), ivec[j] % LANE_TILE, jnp.int32)
            u0 = plsc.load_gather(uslabs.at[slot], [dlo, uc])
            u1 = plsc.load_gather(uslabs.at[slot], [dhi, uc])
            v0 = plsc.load_gather(islabs.at[slot], [dlo, ic])
            v1 = plsc.load_gather(islabs.at[slot], [dhi, ic])
            pltpu.async_copy(slab_src(utabT_hbm, uvecn[j]),
                             uslabs.at[slot], sem_u)
            pltpu.async_copy(slab_src(itabT_hbm, ivecn[j]),
                             islabs.at[slot], sem_i)
            dot = jnp.sum(u0 * v0 + u1 * v1, axis=0)
            acc = acc + dot * onehot[j]
        outb[pl.ds(g * LANES, LANES)] = acc
        return carry

    lax.fori_loop(0, GROUPS, group_body, 0)

    uvecz = uidx[pl.ds(BPW, LANES)]
    for j in range(NRING):
        pltpu.make_async_copy(slab_src(utabT_hbm, uvecz[j]),
                              uslabs.at[j], sem_u).wait()
        pltpu.make_async_copy(slab_src(itabT_hbm, uvecz[j]),
                              islabs.at[j], sem_i).wait()

    pltpu.sync_copy(outb, out_hbm.at[pl.ds(base, BPW)])


def kernel(user_ids, item_ids, user_table, item_table):
    return _sc_dot(user_ids.astype(jnp.int32), item_ids.astype(jnp.int32),
                   user_table.T, item_table.T)
